# baseline (speedup 1.0000x reference)
import functools

import jax
import jax.numpy as jnp
from jax import lax
from jax.experimental import pallas as pl
from jax.experimental.pallas import tpu as pltpu
from jax.experimental.pallas import tpu_sc as plsc

_NW = 32
_C = 64
_SEG = 128
_L = 16
_NBUF = 2


def _build_kernel(n_pad, epi, d):
    K = n_pad // (_NW * _C)
    nseg = (_C * epi) // _SEG
    CE = _C * epi

    mesh = plsc.VectorSubcoreMesh(core_axis_name="c", subcore_axis_name="s")

    scratch = []
    for _ in range(_NBUF):
        scratch += [
            pltpu.VMEM((CE,), jnp.int32),
            pltpu.VMEM((CE,), jnp.float32),
            pltpu.VMEM((CE, d), jnp.float32),
            pltpu.VMEM((_C, d), jnp.float32),
            pltpu.VMEM((_C, d), jnp.float32),
            pltpu.SemaphoreType.DMA,
            pltpu.SemaphoreType.DMA,
            pltpu.SemaphoreType.DMA,
        ]

    @functools.partial(
        pl.kernel,
        out_type=jax.ShapeDtypeStruct((n_pad, d), jnp.float32),
        mesh=mesh,
        compiler_params=pltpu.CompilerParams(use_tc_tiling_on_sc=False),
        scratch_types=scratch,
    )
    def pooled(table, idxf, attf, out_hbm, *bufs):
        idx_v, att_v, rows_v, base_v, out_v, semI, semA, semO = (
            [bufs[b * 8 + i] for b in range(_NBUF)] for i in range(8))
        wid = lax.axis_index("s") * 2 + lax.axis_index("c")
        g0 = wid * K

        def fire_idx(g, b):
            pltpu.async_copy(idxf.at[pl.ds(g * CE, CE)], idx_v[b], semI[b])

        def wait_idx(b):
            pltpu.make_async_copy(
                idxf.at[pl.ds(0, CE)], idx_v[b], semI[b]).wait()

        def fire_stage2(g, b):
            for j in range(nseg):
                pltpu.async_copy(
                    table.at[idx_v[b].at[pl.ds(j * _SEG, _SEG)]],
                    rows_v[b].at[pl.ds(j * _SEG, _SEG)], semA[b])
            pltpu.async_copy(attf.at[pl.ds(g * CE, CE)], att_v[b], semA[b])
            pltpu.async_copy(table.at[pl.ds(g * _C, _C)], base_v[b], semA[b])

        def wait_stage2(b):
            pltpu.make_async_copy(
                table.at[pl.ds(0, CE)], rows_v[b], semA[b]).wait()
            pltpu.make_async_copy(
                attf.at[pl.ds(0, CE)], att_v[b], semA[b]).wait()
            pltpu.make_async_copy(
                table.at[pl.ds(0, _C)], base_v[b], semA[b]).wait()

        def fire_out(g, b):
            pltpu.async_copy(out_v[b], out_hbm.at[pl.ds(g * _C, _C)], semO[b])

        def wait_out(g, b):
            pltpu.make_async_copy(
                out_v[b], out_hbm.at[pl.ds(g * _C, _C)], semO[b]).wait()

        def compute(g, b):
            av, rv, bv, ov = att_v[b], rows_v[b], base_v[b], out_v[b]

            def block_body(ib, carry):
                i0 = ib * _L
                logits = [av[pl.ds(e * _C + i0, _L)] for e in range(epi)]
                m = functools.reduce(jnp.maximum, logits)
                probs = [jnp.exp(v - m) for v in logits]
                s = functools.reduce(jnp.add, probs)
                inv = 1.0 / s
                w = [p * inv for p in probs]
                for lane in range(_L):
                    i = i0 + lane
                    a0 = bv[i, pl.ds(0, _L)]
                    a1 = bv[i, pl.ds(_L, _L)]
                    r = i * epi
                    for e in range(epi):
                        ws = w[e][lane]
                        a0 = a0 + ws * rv[r + e, pl.ds(0, _L)]
                        a1 = a1 + ws * rv[r + e, pl.ds(_L, _L)]
                    ov[i, pl.ds(0, _L)] = a0
                    ov[i, pl.ds(_L, _L)] = a1
                return carry

            lax.fori_loop(0, _C // _L, block_body, 0)

        for b in range(_NBUF):
            fire_idx(g0 + b, b)
        for b in range(_NBUF):
            wait_idx(b)
            fire_stage2(g0 + b, b)

        def outer(u, carry):
            for b in range(_NBUF):
                t = u * _NBUF + b
                g = g0 + t
                wait_stage2(b)

                @pl.when(t < K - _NBUF)
                def _():
                    fire_idx(g + _NBUF, b)

                @pl.when(t >= _NBUF)
                def _():
                    wait_out(g - _NBUF, b)

                compute(g, b)
                fire_out(g, b)

                @pl.when(t < K - _NBUF)
                def _():
                    wait_idx(b)
                    fire_stage2(g + _NBUF, b)

            return carry

        lax.fori_loop(0, K // _NBUF, outer, 0)

        for b in range(_NBUF):
            wait_out(g0 + K - _NBUF + b, b)

    return pooled


def kernel(uEmbeds, entiEmbs, att, item_entities):
    n, epi = att.shape
    d = entiEmbs.shape[1]
    per_round = _NW * _C * _NBUF
    n_pad = ((n + per_round - 1) // per_round) * per_round
    pad = n_pad - n

    idx_flat = item_entities.astype(jnp.int32).reshape(-1)
    att_pad = att.astype(jnp.float32)
    if pad:
        idx_flat = jnp.pad(idx_flat, (0, pad * epi))
        att_pad = jnp.pad(att_pad, ((0, pad), (0, 0)))
    att_flat = att_pad.reshape(-1, _C, epi).transpose(0, 2, 1).reshape(-1)

    pooled = _build_kernel(n_pad, epi, d)
    out = pooled(entiEmbs, idx_flat, att_flat)
    return (uEmbeds, out[:n])


# speedup vs baseline: 1.6775x; 1.6775x over previous
import functools

import jax
import jax.numpy as jnp
from jax import lax
from jax.experimental import pallas as pl
from jax.experimental.pallas import tpu as pltpu
from jax.experimental.pallas import tpu_sc as plsc

_NW = 32
_C = 128
_SEG = 128
_L = 16


def _build_kernel(n, epi, d):
    G = -(-n // _C)
    K = -(-G // _NW)
    nseg = (_C * epi) // _SEG
    CE = _C * epi

    mesh = plsc.VectorSubcoreMesh(core_axis_name="c", subcore_axis_name="s")

    @functools.partial(
        pl.kernel,
        out_type=jax.ShapeDtypeStruct((n, d), jnp.float32),
        mesh=mesh,
        compiler_params=pltpu.CompilerParams(use_tc_tiling_on_sc=False),
        scratch_types=[
            pltpu.VMEM((CE,), jnp.int32),
            pltpu.VMEM((CE,), jnp.float32),
            pltpu.VMEM((CE, d), jnp.float32),
            pltpu.VMEM((_C, d), jnp.float32),
            pltpu.VMEM((_C, d), jnp.float32),
            pltpu.SemaphoreType.DMA,
        ],
    )
    def pooled(table, idxf, attf, out_hbm,
               idx_v, att_v, rows_v, base_v, out_v, semA):
        wid = lax.axis_index("s") * 2 + lax.axis_index("c")

        def chunk_body(t, carry):
            g = wid * K + t
            base = jnp.minimum(g * _C, n - _C)

            pltpu.sync_copy(idxf.at[pl.ds(base * epi, CE)], idx_v)
            descs = []
            for j in range(nseg):
                descs.append(pltpu.async_copy(
                    table.at[idx_v.at[pl.ds(j * _SEG, _SEG)]],
                    rows_v.at[pl.ds(j * _SEG, _SEG)], semA))
            descs.append(pltpu.async_copy(
                attf.at[pl.ds(base * epi, CE)], att_v, semA))
            descs.append(pltpu.async_copy(
                table.at[pl.ds(base, _C)], base_v, semA))
            for desc in descs:
                desc.wait()

            def item_body(i, carry2):
                r = i * epi
                pa = jnp.exp(att_v[pl.ds(r, _L)])
                pb = jnp.exp(att_v[pl.ds(r + epi - _L, _L)])
                a0 = jnp.zeros((_L,), jnp.float32)
                a1 = jnp.zeros((_L,), jnp.float32)
                s = None
                for e in range(epi):
                    src, lane = (pa, e) if e < _L else (pb, e - (epi - _L))
                    wv = lax.broadcast_in_dim(src[lane], (_L,), ())
                    s = wv if s is None else s + wv
                    a0 = a0 + wv * rows_v[r + e, pl.ds(0, _L)]
                    a1 = a1 + wv * rows_v[r + e, pl.ds(_L, _L)]
                inv = 1.0 / s
                out_v[i, pl.ds(0, _L)] = base_v[i, pl.ds(0, _L)] + a0 * inv
                out_v[i, pl.ds(_L, _L)] = base_v[i, pl.ds(_L, _L)] + a1 * inv
                return carry2

            lax.fori_loop(0, _C, item_body, 0)

            pltpu.sync_copy(out_v, out_hbm.at[pl.ds(base, _C)])
            return carry

        lax.fori_loop(0, K, chunk_body, 0)

    return pooled


def kernel(uEmbeds, entiEmbs, att, item_entities):
    n, epi = att.shape
    d = entiEmbs.shape[1]
    idx_flat = item_entities.astype(jnp.int32).reshape(-1)
    att_flat = att.reshape(-1)
    pooled = _build_kernel(n, epi, d)
    out = pooled(entiEmbs, idx_flat, att_flat)
    return (uEmbeds, out)


# speedup vs baseline: 1.8688x; 1.1140x over previous
import functools

import jax
import jax.numpy as jnp
from jax import lax
from jax.experimental import pallas as pl
from jax.experimental.pallas import tpu as pltpu
from jax.experimental.pallas import tpu_sc as plsc

_NW = 32
_C = 64
_SEG = 128
_L = 16
_NBUF = 2


def _build_kernel(n, epi, d):
    G = -(-n // _C)
    K = -(-G // (_NW * _NBUF)) * _NBUF
    nseg = (_C * epi) // _SEG
    CE = _C * epi

    mesh = plsc.VectorSubcoreMesh(core_axis_name="c", subcore_axis_name="s")

    scratch = []
    for _ in range(_NBUF):
        scratch += [
            pltpu.VMEM((CE,), jnp.int32),
            pltpu.VMEM((CE,), jnp.float32),
            pltpu.VMEM((CE, d), jnp.float32),
            pltpu.VMEM((_C, d), jnp.float32),
            pltpu.VMEM((_C, d), jnp.float32),
            pltpu.SemaphoreType.DMA,
            pltpu.SemaphoreType.DMA,
            pltpu.SemaphoreType.DMA,
        ]

    @functools.partial(
        pl.kernel,
        out_type=jax.ShapeDtypeStruct((n, d), jnp.float32),
        mesh=mesh,
        compiler_params=pltpu.CompilerParams(use_tc_tiling_on_sc=False),
        scratch_types=scratch,
    )
    def pooled(table, idxf, attf, out_hbm, *bufs):
        idx_v, att_v, rows_v, base_v, out_v, semI, semA, semO = (
            [bufs[b * 8 + i] for b in range(_NBUF)] for i in range(8))
        wid = lax.axis_index("s") * 2 + lax.axis_index("c")
        g0 = wid * K
        last = n - _C

        def chunk_base(t):
            return jnp.minimum((g0 + t) * _C, last)

        def fire_idx(t, b):
            base = chunk_base(t)
            pltpu.async_copy(
                idxf.at[pl.ds(base * epi, CE)], idx_v[b], semI[b])

        def wait_idx(b):
            pltpu.make_async_copy(
                idxf.at[pl.ds(0, CE)], idx_v[b], semI[b]).wait()

        def fire_stage2(t, b):
            base = chunk_base(t)
            for j in range(nseg):
                pltpu.async_copy(
                    table.at[idx_v[b].at[pl.ds(j * _SEG, _SEG)]],
                    rows_v[b].at[pl.ds(j * _SEG, _SEG)], semA[b])
            pltpu.async_copy(
                attf.at[pl.ds(base * epi, CE)], att_v[b], semA[b])
            pltpu.async_copy(table.at[pl.ds(base, _C)], base_v[b], semA[b])

        def wait_stage2(b):
            pltpu.make_async_copy(
                table.at[pl.ds(0, CE)], rows_v[b], semA[b]).wait()
            pltpu.make_async_copy(
                attf.at[pl.ds(0, CE)], att_v[b], semA[b]).wait()
            pltpu.make_async_copy(
                table.at[pl.ds(0, _C)], base_v[b], semA[b]).wait()

        def fire_out(t, b):
            base = chunk_base(t)
            pltpu.async_copy(
                out_v[b], out_hbm.at[pl.ds(base, _C)], semO[b])

        def wait_out(b):
            pltpu.make_async_copy(
                out_v[b], out_hbm.at[pl.ds(0, _C)], semO[b]).wait()

        def compute(b):
            av, rv, bv, ov = att_v[b], rows_v[b], base_v[b], out_v[b]

            def item_body(i, carry2):
                r = i * epi
                pa = jnp.exp(av[pl.ds(r, _L)])
                pb = jnp.exp(av[pl.ds(r + epi - _L, _L)])
                a0 = jnp.zeros((_L,), jnp.float32)
                a1 = jnp.zeros((_L,), jnp.float32)
                s = None
                for e in range(epi):
                    src, lane = (pa, e) if e < _L else (pb, e - (epi - _L))
                    wv = lax.broadcast_in_dim(src[lane], (_L,), ())
                    s = wv if s is None else s + wv
                    a0 = a0 + wv * rv[r + e, pl.ds(0, _L)]
                    a1 = a1 + wv * rv[r + e, pl.ds(_L, _L)]
                inv = 1.0 / s
                ov[i, pl.ds(0, _L)] = bv[i, pl.ds(0, _L)] + a0 * inv
                ov[i, pl.ds(_L, _L)] = bv[i, pl.ds(_L, _L)] + a1 * inv
                return carry2

            lax.fori_loop(0, _C, item_body, 0)

        for b in range(_NBUF):
            fire_idx(b, b)
        for b in range(_NBUF):
            wait_idx(b)
            fire_stage2(b, b)
            fire_out(K - _NBUF + b, b)

        def outer(u, carry):
            for b in range(_NBUF):
                t = u * _NBUF + b
                wait_stage2(b)
                fire_idx(jnp.minimum(t + _NBUF, K - 1), b)
                wait_out(b)
                compute(b)
                fire_out(t, b)
                wait_idx(b)
                fire_stage2(jnp.minimum(t + _NBUF, K - 1), b)
            return carry

        lax.fori_loop(0, K // _NBUF, outer, 0)

        for b in range(_NBUF):
            wait_stage2(b)
            wait_out(b)

    return pooled


def kernel(uEmbeds, entiEmbs, att, item_entities):
    n, epi = att.shape
    d = entiEmbs.shape[1]
    idx_flat = item_entities.astype(jnp.int32).reshape(-1)
    att_flat = att.reshape(-1)
    pooled = _build_kernel(n, epi, d)
    out = pooled(entiEmbs, idx_flat, att_flat)
    return (uEmbeds, out)


# speedup vs baseline: 1.8710x; 1.0012x over previous
import functools

import jax
import jax.numpy as jnp
from jax import lax
from jax.experimental import pallas as pl
from jax.experimental.pallas import tpu as pltpu
from jax.experimental.pallas import tpu_sc as plsc

_NW = 32
_C = 64
_SEG = 128
_L = 16
_NBUF = 2


def _build_kernel(n, epi, d):
    G = -(-n // _C)
    K = -(-G // (_NW * _NBUF)) * _NBUF
    nseg = (_C * epi) // _SEG
    CE = _C * epi

    mesh = plsc.VectorSubcoreMesh(core_axis_name="c", subcore_axis_name="s")

    scratch = []
    for _ in range(_NBUF):
        scratch += [
            pltpu.VMEM((CE,), jnp.int32),
            pltpu.VMEM((CE,), jnp.float32),
            pltpu.VMEM((CE, d), jnp.float32),
            pltpu.VMEM((_C, d), jnp.float32),
            pltpu.VMEM((_C, d), jnp.float32),
            pltpu.SemaphoreType.DMA,
            pltpu.SemaphoreType.DMA,
            pltpu.SemaphoreType.DMA,
        ]

    @functools.partial(
        pl.kernel,
        out_type=jax.ShapeDtypeStruct((n, d), jnp.float32),
        mesh=mesh,
        compiler_params=pltpu.CompilerParams(use_tc_tiling_on_sc=False),
        scratch_types=scratch,
    )
    def pooled(table, idxf, attf, out_hbm, *bufs):
        idx_v, att_v, rows_v, base_v, out_v, semI, semA, semO = (
            [bufs[b * 8 + i] for b in range(_NBUF)] for i in range(8))
        wid = lax.axis_index("s") * 2 + lax.axis_index("c")
        g0 = wid * K
        last = n - _C

        def chunk_base(t):
            return jnp.minimum((g0 + t) * _C, last)

        def fire_idx(t, b):
            base = chunk_base(t)
            for e in range(epi):
                pltpu.async_copy(
                    idxf.at[pl.ds(e * n + base, _C)],
                    idx_v[b].at[pl.ds(e * _C, _C)], semI[b])

        def wait_idx(b):
            for e in range(epi):
                pltpu.make_async_copy(
                    idxf.at[pl.ds(0, _C)],
                    idx_v[b].at[pl.ds(e * _C, _C)], semI[b]).wait()

        def fire_stage2(t, b):
            base = chunk_base(t)
            for j in range(nseg):
                pltpu.async_copy(
                    table.at[idx_v[b].at[pl.ds(j * _SEG, _SEG)]],
                    rows_v[b].at[pl.ds(j * _SEG, _SEG)], semA[b])
            for e in range(epi):
                pltpu.async_copy(
                    attf.at[pl.ds(e * n + base, _C)],
                    att_v[b].at[pl.ds(e * _C, _C)], semA[b])
            pltpu.async_copy(table.at[pl.ds(base, _C)], base_v[b], semA[b])

        def wait_stage2(b):
            pltpu.make_async_copy(
                table.at[pl.ds(0, CE)], rows_v[b], semA[b]).wait()
            for e in range(epi):
                pltpu.make_async_copy(
                    attf.at[pl.ds(0, _C)],
                    att_v[b].at[pl.ds(e * _C, _C)], semA[b]).wait()
            pltpu.make_async_copy(
                table.at[pl.ds(0, _C)], base_v[b], semA[b]).wait()

        def fire_out(t, b):
            base = chunk_base(t)
            pltpu.async_copy(
                out_v[b], out_hbm.at[pl.ds(base, _C)], semO[b])

        def wait_out(b):
            pltpu.make_async_copy(
                out_v[b], out_hbm.at[pl.ds(0, _C)], semO[b]).wait()

        def compute(b):
            av, rv, bv, ov = att_v[b], rows_v[b], base_v[b], out_v[b]

            def block_body(ib, carry2):
                i0 = ib * _L
                probs = [jnp.exp(av[pl.ds(e * _C + i0, _L)])
                         for e in range(epi)]
                s = functools.reduce(jnp.add, probs)
                inv = 1.0 / s
                w = [p * inv for p in probs]
                for lane in range(_L):
                    i = i0 + lane
                    a0 = bv[i, pl.ds(0, _L)]
                    a1 = bv[i, pl.ds(_L, _L)]
                    for e in range(epi):
                        wv = lax.broadcast_in_dim(w[e][lane], (_L,), ())
                        a0 = a0 + wv * rv[e * _C + i, pl.ds(0, _L)]
                        a1 = a1 + wv * rv[e * _C + i, pl.ds(_L, _L)]
                    ov[i, pl.ds(0, _L)] = a0
                    ov[i, pl.ds(_L, _L)] = a1
                return carry2

            lax.fori_loop(0, _C // _L, block_body, 0)

        for b in range(_NBUF):
            fire_idx(b, b)
        for b in range(_NBUF):
            wait_idx(b)
            fire_stage2(b, b)
            fire_out(K - _NBUF + b, b)

        def outer(u, carry):
            for b in range(_NBUF):
                t = u * _NBUF + b
                wait_stage2(b)
                fire_idx(jnp.minimum(t + _NBUF, K - 1), b)
                wait_out(b)
                compute(b)
                fire_out(t, b)
                wait_idx(b)
                fire_stage2(jnp.minimum(t + _NBUF, K - 1), b)
            return carry

        lax.fori_loop(0, K // _NBUF, outer, 0)

        for b in range(_NBUF):
            wait_stage2(b)
            wait_out(b)

    return pooled


def kernel(uEmbeds, entiEmbs, att, item_entities):
    n, epi = att.shape
    d = entiEmbs.shape[1]
    idx_flat = item_entities.T.astype(jnp.int32).reshape(-1)
    att_flat = att.T.reshape(-1)
    pooled = _build_kernel(n, epi, d)
    out = pooled(entiEmbs, idx_flat, att_flat)
    return (uEmbeds, out)
